# baseline (speedup 1.0000x reference)
import functools

import jax
import jax.numpy as jnp
from jax import lax
from jax.experimental import pallas as pl
from jax.experimental.pallas import tpu as pltpu
from jax.experimental.pallas import tpu_sc as plsc

T, H, W, C = 4, 56, 56, 384
HD, K = 8, 25
Q = T * H * W
F = C // HD
R = Q * HD
KP = 32

NC, NS, NL = 2, 16, 16
NW = NC * NS
FR = H * W
BH = 4
RB = 8
NB = H // BH
RPB = BH * W
WROWS = T * RB * W
WQ = 32767
PKR = RPB * KP // 128


def _mm_kernel(x_ref, w_ref, o_ref):
    o_ref[...] = jnp.dot(x_ref[...], w_ref[...],
                         preferred_element_type=jnp.float32)


def _mm(x, w, bm):
    m = x.shape[0]
    return pl.pallas_call(
        _mm_kernel,
        grid=(m // bm,),
        in_specs=[
            pl.BlockSpec((bm, x.shape[1]), lambda i: (i, 0)),
            pl.BlockSpec(w.shape, lambda i: (0, 0)),
        ],
        out_specs=pl.BlockSpec((bm, w.shape[1]), lambda i: (i, 0)),
        out_shape=jax.ShapeDtypeStruct((m, w.shape[1]), jnp.float32),
    )(x, w)


def _mm2_kernel(a_ref, w_ref, b_ref, o_ref):
    o_ref[...] = jnp.dot(a_ref[...], w_ref[...],
                         preferred_element_type=jnp.float32) + b_ref[...]


def _mm2(agg, wp, bp, bm):
    return pl.pallas_call(
        _mm2_kernel,
        grid=(Q // bm,),
        in_specs=[
            pl.BlockSpec((bm, C), lambda m: (m, 0)),
            pl.BlockSpec((C, C), lambda m: (0, 0)),
            pl.BlockSpec((1, C), lambda m: (0, 0)),
        ],
        out_specs=pl.BlockSpec((bm, C), lambda m: (m, 0)),
        out_shape=jax.ShapeDtypeStruct((Q, C), jnp.float32),
    )(agg, wp, bp.reshape(1, C))


def _sc_agg_body(vtab, pk_hbm, out_hbm, win, pk_v, out_v, sem, osem):
    wid = lax.axis_index("s") * NC + lax.axis_index("c")
    hd = wid // T
    t = wid % T
    qbase = t * FR
    invwq = jnp.float32(1.0 / WQ)

    def stage_rows(r0, n, cps):
        for tp in range(T):
            src = vtab.at[pl.ds(tp * FR + r0 * W, n * W), pl.ds(hd * F, F)]
            dst = win.at[pl.ds((tp * RB + (r0 % RB)) * W, n * W), :]
            cps.append(pltpu.async_copy(src, dst, sem))

    def make_qbody(pk_b, out_b):
        def qbody(r, carry):
            row = r // 4
            lane = (r % 4) * KP
            pv0 = pk_b[row, pl.ds(lane, NL)]
            pv1 = pk_b[row, pl.ds(lane + K - NL, NL)]
            ix0 = pv0 & 2047
            ix1 = pv1 & 2047
            wv0 = lax.shift_right_logical(pv0, 11).astype(jnp.float32)
            wv1 = lax.shift_right_logical(pv1, 11).astype(jnp.float32)
            acc = [jnp.zeros((NL,), jnp.float32) for _ in range(6)]
            for k in range(K):
                if k < NL:
                    ix = ix0[k]
                    wgt = wv0[k]
                else:
                    ix = ix1[k - (K - NL)]
                    wgt = wv1[k - (K - NL)]
                p = 3 * (k & 1)
                acc[p] = acc[p] + wgt * win[ix, pl.ds(0, NL)]
                acc[p + 1] = acc[p + 1] + wgt * win[ix, pl.ds(NL, NL)]
                acc[p + 2] = acc[p + 2] + wgt * win[ix, pl.ds(2 * NL, NL)]
            out_b[r, pl.ds(0, NL)] = (acc[0] + acc[3]) * invwq
            out_b[r, pl.ds(NL, NL)] = (acc[1] + acc[4]) * invwq
            out_b[r, pl.ds(2 * NL, NL)] = (acc[2] + acc[5]) * invwq
            return carry
        return qbody

    cps = []
    stage_rows(0, 2, cps)
    cps.append(pltpu.async_copy(
        pk_hbm.at[hd, pl.ds(qbase * KP // 128, PKR), :], pk_v.at[0], sem))
    out_cps = []
    for b in range(NB):
        lo = BH * b + 2
        hi = min(BH * b + BH + 1, H - 1)
        r = lo
        while r <= hi:
            rend = min(hi, r + (RB - 1 - (r % RB)))
            stage_rows(r, rend - r + 1, cps)
            r = rend + 1
        qb = qbase + b * RPB
        if b + 1 < NB:
            cps.append(pltpu.async_copy(
                pk_hbm.at[hd, pl.ds((qb + RPB) * KP // 128, PKR), :],
                pk_v.at[(b + 1) % 2], sem))
        for cp in cps:
            cp.wait()
        cps = []
        if b >= 2:
            out_cps[b - 2].wait()
        lax.fori_loop(0, RPB,
                      make_qbody(pk_v.at[b % 2], out_v.at[b % 2]), 0)
        out_cps.append(pltpu.async_copy(
            out_v.at[b % 2],
            out_hbm.at[pl.ds(qb, RPB), pl.ds(hd * F, F)], osem))
    out_cps[NB - 2].wait()
    out_cps[NB - 1].wait()


def _sc_agg(vtab, packed):
    mesh = plsc.VectorSubcoreMesh(core_axis_name="c", subcore_axis_name="s")
    kern = functools.partial(
        pl.kernel,
        out_type=jax.ShapeDtypeStruct((Q, C), jnp.float32),
        mesh=mesh,
        scratch_types=[
            pltpu.VMEM((WROWS, F), jnp.float32),
            pltpu.VMEM((2, PKR, 128), jnp.int32),
            pltpu.VMEM((2, RPB, F), jnp.float32),
            pltpu.SemaphoreType.DMA,
            pltpu.SemaphoreType.DMA,
        ],
        compiler_params=pltpu.CompilerParams(use_tc_tiling_on_sc=False),
    )(_sc_agg_body)
    return kern(vtab, packed)


def kernel(x, attn, flows, Wv, Wp, bp):
    x2d = x.reshape(Q, C)

    q = jnp.arange(Q, dtype=jnp.int32)
    tq = q // FR
    hq = (q // W) % H
    wq = q % W
    fl = flows[0]
    tt = jnp.clip(tq[None, :, None] + fl[..., 0], 0, T - 1)
    hh = jnp.clip(hq[None, :, None] + fl[..., 1], 0, H - 1)
    ww = jnp.clip(wq[None, :, None] + fl[..., 2], 0, W - 1)
    widx = (tt * RB + hh % RB) * W + ww
    wq15 = (attn[0] * WQ + 0.5).astype(jnp.int32)
    packed = jnp.concatenate(
        [widx | (wq15 << 11),
         jnp.zeros((HD, Q, KP - K), jnp.int32)], axis=2)
    packed = packed.reshape(HD, Q * KP // 128, 128)

    v = _mm(x2d, Wv, 1568)

    agg = _sc_agg(v, packed)

    y = _mm2(agg, Wp, bp, 1568)
    return y.reshape(T, H, W, C)
